# baseline (speedup 1.0000x reference)
import jax
import jax.numpy as jnp
from jax.experimental import pallas as pl
from jax.experimental.pallas import tpu as pltpu

N = 4096
REL = 2
NB = 2
H0 = 64
H1 = 64
BLK = 512
NBLK = N // BLK


def _rgcn_body(c0_ref, c1_ref, adj_ref, bw0_ref, bw1_ref,
               out1_ref, fsum_ref, out0_scr, w0_scr):
    p = pl.program_id(0)
    i = pl.program_id(1)

    @pl.when(jnp.logical_and(p == 0, i == 0))
    def _init():
        fsum_ref[...] = jnp.zeros_like(fsum_ref)
        for r in range(REL):
            w0_scr[r] = c0_ref[r, 0] * bw0_ref[0] + c0_ref[r, 1] * bw0_ref[1]

    @pl.when(p == 0)
    def _phase0():
        acc = jnp.zeros((BLK, H0), dtype=jnp.float32)
        for r in range(REL):
            acc = acc + jnp.dot(adj_ref[r], w0_scr[r],
                                preferred_element_type=jnp.float32)
        out0_scr[pl.ds(i * BLK, BLK), :] = acc
        fsum_ref[0:1, :] = fsum_ref[0:1, :] + jnp.sum(acc, axis=0, keepdims=True)

    @pl.when(p == 1)
    def _phase1():
        h = jnp.maximum(out0_scr[...], 0.0)
        acc = jnp.zeros((BLK, H1), dtype=jnp.float32)
        for r in range(REL):
            t = jnp.dot(adj_ref[r], h, preferred_element_type=jnp.float32)
            w1 = c1_ref[r, 0] * bw1_ref[0] + c1_ref[r, 1] * bw1_ref[1]
            acc = acc + jnp.dot(t, w1, preferred_element_type=jnp.float32)
        out1_ref[...] = acc
        fsum_ref[1:2, :] = fsum_ref[1:2, :] + jnp.sum(acc, axis=0, keepdims=True)


def kernel(adj, basis_weight0, basis_coeff0, basis_weight1, basis_coeff1):
    out1, fsum = pl.pallas_call(
        _rgcn_body,
        grid=(2, NBLK),
        in_specs=[
            pl.BlockSpec(memory_space=pltpu.SMEM),
            pl.BlockSpec(memory_space=pltpu.SMEM),
            pl.BlockSpec((REL, BLK, N), lambda p, i: (0, i, 0)),
            pl.BlockSpec((NB, N, H0), lambda p, i: (0, 0, 0)),
            pl.BlockSpec((NB, H0, H1), lambda p, i: (0, 0, 0)),
        ],
        out_specs=[
            pl.BlockSpec((BLK, H1), lambda p, i: (i, 0)),
            pl.BlockSpec((2, H0), lambda p, i: (0, 0)),
        ],
        out_shape=[
            jax.ShapeDtypeStruct((N, H1), jnp.float32),
            jax.ShapeDtypeStruct((2, H0), jnp.float32),
        ],
        scratch_shapes=[
            pltpu.VMEM((N, H0), jnp.float32),
            pltpu.VMEM((REL, N, H0), jnp.float32),
        ],
        compiler_params=pltpu.CompilerParams(
            dimension_semantics=("arbitrary", "arbitrary")),
    )(basis_coeff0, basis_coeff1, adj, basis_weight0, basis_weight1)
    final = fsum.reshape(1, H0 + H1)
    return (out1, final)


# speedup vs baseline: 1.0094x; 1.0094x over previous
import jax
import jax.numpy as jnp
from jax.experimental import pallas as pl
from jax.experimental.pallas import tpu as pltpu

N = 4096
REL = 2
NB = 2
H0 = 64
H1 = 64
BLK = 256
NBLK = N // BLK


def _rgcn_body(c0_ref, c1_ref, adj_ref, bw0_ref, bw1_ref,
               out1_ref, fsum_ref, out0_scr, w0_scr):
    p = pl.program_id(0)
    i = pl.program_id(1)

    @pl.when(jnp.logical_and(p == 0, i == 0))
    def _init():
        fsum_ref[...] = jnp.zeros_like(fsum_ref)
        for r in range(REL):
            w0_scr[r] = c0_ref[r, 0] * bw0_ref[0] + c0_ref[r, 1] * bw0_ref[1]

    @pl.when(p == 0)
    def _phase0():
        acc = jnp.zeros((BLK, H0), dtype=jnp.float32)
        for r in range(REL):
            acc = acc + jnp.dot(adj_ref[r].astype(jnp.bfloat16),
                                w0_scr[r].astype(jnp.bfloat16),
                                preferred_element_type=jnp.float32)
        out0_scr[pl.ds(i * BLK, BLK), :] = acc
        fsum_ref[0:1, :] = fsum_ref[0:1, :] + jnp.sum(acc, axis=0, keepdims=True)

    @pl.when(p == 1)
    def _phase1():
        h = jnp.maximum(out0_scr[...], 0.0).astype(jnp.bfloat16)
        acc = jnp.zeros((BLK, H1), dtype=jnp.float32)
        for r in range(REL):
            t = jnp.dot(adj_ref[r].astype(jnp.bfloat16), h,
                        preferred_element_type=jnp.float32)
            w1 = c1_ref[r, 0] * bw1_ref[0] + c1_ref[r, 1] * bw1_ref[1]
            acc = acc + jnp.dot(t, w1, preferred_element_type=jnp.float32)
        out1_ref[...] = acc
        fsum_ref[1:2, :] = fsum_ref[1:2, :] + jnp.sum(acc, axis=0, keepdims=True)


def kernel(adj, basis_weight0, basis_coeff0, basis_weight1, basis_coeff1):
    out1, fsum = pl.pallas_call(
        _rgcn_body,
        grid=(2, NBLK),
        in_specs=[
            pl.BlockSpec(memory_space=pltpu.SMEM),
            pl.BlockSpec(memory_space=pltpu.SMEM),
            pl.BlockSpec((REL, BLK, N), lambda p, i: (0, i, 0)),
            pl.BlockSpec((NB, N, H0), lambda p, i: (0, 0, 0)),
            pl.BlockSpec((NB, H0, H1), lambda p, i: (0, 0, 0)),
        ],
        out_specs=[
            pl.BlockSpec((BLK, H1), lambda p, i: (i, 0)),
            pl.BlockSpec((2, H0), lambda p, i: (0, 0)),
        ],
        out_shape=[
            jax.ShapeDtypeStruct((N, H1), jnp.float32),
            jax.ShapeDtypeStruct((2, H0), jnp.float32),
        ],
        scratch_shapes=[
            pltpu.VMEM((N, H0), jnp.float32),
            pltpu.VMEM((REL, N, H0), jnp.float32),
        ],
        compiler_params=pltpu.CompilerParams(
            dimension_semantics=("arbitrary", "arbitrary")),
    )(basis_coeff0, basis_coeff1, adj, basis_weight0, basis_weight1)
    final = fsum.reshape(1, H0 + H1)
    return (out1, final)
